# baseline (speedup 1.0000x reference)
import functools

import jax
import jax.numpy as jnp
from jax import lax
from jax.experimental import pallas as pl
from jax.experimental.pallas import tpu as pltpu
from jax.experimental.pallas import tpu_sc as plsc

N = 10000
E = 320000
F_IN = 128
HID = 128
CLS = 40

NC = 2
NS = 16
NW = NC * NS

CHUNK = 128
ROWS_PER_WORKER = 80
E_PAD = NW * ROWS_PER_WORKER * CHUNK
N_PAD = N + 8

TILE_ROWS = 624
REM_ROWS = N - TILE_ROWS * NS
REM_R0 = TILE_ROWS * NS

BLK = 1024


def _sc_mesh():
    return plsc.VectorSubcoreMesh(core_axis_name="c", subcore_axis_name="s")


E_PER_W = E_PAD // NW
HIST_PAD = 10240


@functools.partial(
    pl.kernel,
    mesh=_sc_mesh(),
    compiler_params=pltpu.CompilerParams(use_tc_tiling_on_sc=False,
                                         needs_layout_passes=False),
    out_type=jax.ShapeDtypeStruct((NW, HIST_PAD), jnp.float32),
    scratch_types=[
        pltpu.VMEM((E_PER_W,), jnp.int32),
        pltpu.VMEM((HIST_PAD,), jnp.float32),
    ],
)
def _sc_degree(dst_hbm, out_hbm, dst_v, hist):
    cid = lax.axis_index("c")
    sid = lax.axis_index("s")
    wid = cid * NS + sid

    @pl.loop(0, HIST_PAD // 16)
    def _(i):
        hist[pl.ds(i * 16, 16)] = jnp.zeros((16,), jnp.float32)

    pltpu.sync_copy(dst_hbm.at[wid], dst_v)
    one16 = jnp.full((16,), 1.0, jnp.float32)

    @pl.loop(0, E_PER_W // 16)
    def _(k):
        idx = dst_v[pl.ds(k * 16, 16)]
        plsc.addupdate_scatter(hist, [idx], one16)

    pltpu.sync_copy(hist, out_hbm.at[wid])


def _make_sc_aggregate(d):
    @functools.partial(
        pl.kernel,
        mesh=_sc_mesh(),
        compiler_params=pltpu.CompilerParams(use_tc_tiling_on_sc=False),
        out_type=jax.ShapeDtypeStruct((NC * N, d), jnp.float32),
        scratch_types=[
            pltpu.VMEM((ROWS_PER_WORKER, CHUNK), jnp.int32),
            pltpu.VMEM((ROWS_PER_WORKER, CHUNK), jnp.int32),
            pltpu.VMEM((CHUNK, d), jnp.float32),
            pltpu.VMEM((CHUNK, d), jnp.float32),
            pltpu.VMEM_SHARED((N_PAD, d), jnp.float32),
            pltpu.SemaphoreType.DMA,
            pltpu.SemaphoreType.DMA,
        ],
    )
    def agg(y_hbm, src_hbm, dst_hbm, out_hbm, src_v, dst_v, rows_a, rows_b,
            acc_sh, sem_a, sem_b):
        cid = lax.axis_index("c")
        sid = lax.axis_index("s")
        wid = cid * NS + sid
        r0 = sid * TILE_ROWS
        pltpu.sync_copy(y_hbm.at[pl.ds(r0, TILE_ROWS)],
                        acc_sh.at[pl.ds(r0, TILE_ROWS)])

        @pl.when(sid == NS - 1)
        def _():
            pltpu.sync_copy(y_hbm.at[pl.ds(REM_R0, REM_ROWS + 8)],
                            acc_sh.at[pl.ds(REM_R0, REM_ROWS + 8)])

        pltpu.sync_copy(src_hbm.at[wid], src_v)
        pltpu.sync_copy(dst_hbm.at[wid], dst_v)
        plsc.subcore_barrier()

        bufs = (rows_a, rows_b)
        sems = (sem_a, sem_b)

        def gather(j, b):
            pltpu.async_copy(y_hbm.at[src_v.at[j]], bufs[b], sems[b])

        def gwait(b):
            pltpu.make_async_copy(y_hbm.at[src_v.at[0]], bufs[b],
                                  sems[b]).wait()

        gather(0, 0)

        @pl.loop(0, ROWS_PER_WORKER // 2)
        def _(i):
            j0 = 2 * i
            gather(j0 + 1, 1)
            gwait(0)
            pltpu.sync_copy(bufs[0], acc_sh.at[dst_v.at[j0]], add=True)
            gather(lax.rem(j0 + 2, ROWS_PER_WORKER), 0)
            gwait(1)
            pltpu.sync_copy(bufs[1], acc_sh.at[dst_v.at[j0 + 1]], add=True)

        gwait(0)

        plsc.subcore_barrier()
        pltpu.sync_copy(acc_sh.at[pl.ds(r0, TILE_ROWS)],
                        out_hbm.at[pl.ds(cid * N + r0, TILE_ROWS)])

        @pl.when(sid == NS - 1)
        def _():
            pltpu.sync_copy(acc_sh.at[pl.ds(REM_R0, REM_ROWS)],
                            out_hbm.at[pl.ds(cid * N + REM_R0, REM_ROWS)])

    return agg


_sc_agg_64 = _make_sc_aggregate(64)
_sc_agg_cls = _make_sc_aggregate(CLS)


def _tc1_body(dp_ref, x_ref, w1_ref, y1a_ref, y1b_ref, dinv_ref):
    deg = jnp.sum(dp_ref[...], axis=0, keepdims=True) + 1.0
    dinv = lax.rsqrt(deg).reshape(BLK, 1)
    xw = jnp.dot(x_ref[...], w1_ref[...], preferred_element_type=jnp.float32)
    y1 = dinv * xw
    y1a_ref[...] = y1[:, :64]
    y1b_ref[...] = y1[:, 64:]
    dinv_ref[...] = jnp.broadcast_to(dinv, dinv_ref.shape)


_tc1 = pl.pallas_call(
    _tc1_body,
    grid=(pl.cdiv(N, BLK),),
    in_specs=[
        pl.BlockSpec((NW, BLK), lambda i: (0, i)),
        pl.BlockSpec((BLK, F_IN), lambda i: (i, 0)),
        pl.BlockSpec((F_IN, HID), lambda i: (0, 0)),
    ],
    out_specs=[
        pl.BlockSpec((BLK, 64), lambda i: (i, 0)),
        pl.BlockSpec((BLK, 64), lambda i: (i, 0)),
        pl.BlockSpec((BLK, 8), lambda i: (i, 0)),
    ],
    out_shape=[
        jax.ShapeDtypeStruct((N_PAD, 64), jnp.float32),
        jax.ShapeDtypeStruct((N_PAD, 64), jnp.float32),
        jax.ShapeDtypeStruct((N, 8), jnp.float32),
    ],
)


def _tc2_body(paa_ref, pab_ref, pba_ref, pbb_ref, ya_ref, yb_ref,
              dinv_ref, b1_ref, w2_ref, y2_ref):
    dinv = dinv_ref[:, :1]
    a1 = jnp.concatenate(
        [paa_ref[...] + pab_ref[...] - ya_ref[...],
         pba_ref[...] + pbb_ref[...] - yb_ref[...]], axis=1)
    h = jnp.maximum(dinv * a1 + b1_ref[...], 0.0)
    y2_ref[...] = dinv * jnp.dot(h, w2_ref[...],
                                 preferred_element_type=jnp.float32)


_tc2 = pl.pallas_call(
    _tc2_body,
    grid=(pl.cdiv(N, BLK),),
    in_specs=[
        pl.BlockSpec((BLK, 64), lambda i: (i, 0)),
        pl.BlockSpec((BLK, 64), lambda i: (i, 0)),
        pl.BlockSpec((BLK, 64), lambda i: (i, 0)),
        pl.BlockSpec((BLK, 64), lambda i: (i, 0)),
        pl.BlockSpec((BLK, 64), lambda i: (i, 0)),
        pl.BlockSpec((BLK, 64), lambda i: (i, 0)),
        pl.BlockSpec((BLK, 8), lambda i: (i, 0)),
        pl.BlockSpec((1, HID), lambda i: (0, 0)),
        pl.BlockSpec((HID, CLS), lambda i: (0, 0)),
    ],
    out_specs=pl.BlockSpec((BLK, CLS), lambda i: (i, 0)),
    out_shape=jax.ShapeDtypeStruct((N_PAD, CLS), jnp.float32),
)


def _tc3_body(pa_ref, pb_ref, y2_ref, dinv_ref, b2_ref, out_ref):
    dinv = dinv_ref[:, :1]
    z = dinv * (pa_ref[...] + pb_ref[...] - y2_ref[...]) + b2_ref[...]
    m = jnp.max(z, axis=1, keepdims=True)
    lse = m + jnp.log(jnp.sum(jnp.exp(z - m), axis=1, keepdims=True))
    out_ref[...] = z - lse


_tc3 = pl.pallas_call(
    _tc3_body,
    grid=(pl.cdiv(N, BLK),),
    in_specs=[
        pl.BlockSpec((BLK, CLS), lambda i: (i, 0)),
        pl.BlockSpec((BLK, CLS), lambda i: (i, 0)),
        pl.BlockSpec((BLK, CLS), lambda i: (i, 0)),
        pl.BlockSpec((BLK, 8), lambda i: (i, 0)),
        pl.BlockSpec((1, CLS), lambda i: (0, 0)),
    ],
    out_specs=pl.BlockSpec((BLK, CLS), lambda i: (i, 0)),
    out_shape=jax.ShapeDtypeStruct((N, CLS), jnp.float32),
)


def kernel(x, edge_index, W1, b1, W2, b2):
    ei = edge_index.astype(jnp.int32)
    pad = jnp.full((E_PAD - E,), N, jnp.int32)
    src3d = jnp.concatenate([ei[0], pad]).reshape(NW, ROWS_PER_WORKER, CHUNK)
    dst3d = jnp.concatenate([ei[1], pad]).reshape(NW, ROWS_PER_WORKER, CHUNK)
    dst2d = dst3d.reshape(NW, E_PER_W)

    degp = _sc_degree(dst2d)
    y1a, y1b, dinv8 = _tc1(degp, x, W1)
    p1a = _sc_agg_64(y1a, src3d, dst3d)
    p1b = _sc_agg_64(y1b, src3d, dst3d)
    y2 = _tc2(p1a[:N], p1a[N:], p1b[:N], p1b[N:], y1a, y1b, dinv8,
              b1.reshape(1, HID), W2)
    p2 = _sc_agg_cls(y2, src3d, dst3d)
    return _tc3(p2[:N], p2[N:], y2, dinv8, b2.reshape(1, CLS))


# speedup vs baseline: 1.1064x; 1.1064x over previous
import functools

import jax
import jax.numpy as jnp
from jax import lax
from jax.experimental import pallas as pl
from jax.experimental.pallas import tpu as pltpu
from jax.experimental.pallas import tpu_sc as plsc

N = 10000
E = 320000
F_IN = 128
HID = 128
CLS = 40

NC = 2
NS = 16
NW = NC * NS

CHUNK = 128
ROWS_PER_WORKER = 80
E_PAD = NW * ROWS_PER_WORKER * CHUNK
N_PAD = N + 8

TILE_ROWS = 624
REM_ROWS = N - TILE_ROWS * NS
REM_R0 = TILE_ROWS * NS

BLK = 1024


def _sc_mesh():
    return plsc.VectorSubcoreMesh(core_axis_name="c", subcore_axis_name="s")


E_PER_W = E_PAD // NW
HIST_PAD = 10240


@functools.partial(
    pl.kernel,
    mesh=_sc_mesh(),
    compiler_params=pltpu.CompilerParams(use_tc_tiling_on_sc=False,
                                         needs_layout_passes=False),
    out_type=jax.ShapeDtypeStruct((NW, HIST_PAD), jnp.float32),
    scratch_types=[
        pltpu.VMEM((E_PER_W,), jnp.int32),
        pltpu.VMEM((HIST_PAD,), jnp.float32),
    ],
)
def _sc_degree(dst_hbm, out_hbm, dst_v, hist):
    cid = lax.axis_index("c")
    sid = lax.axis_index("s")
    wid = cid * NS + sid

    @pl.loop(0, HIST_PAD // 16)
    def _(i):
        hist[pl.ds(i * 16, 16)] = jnp.zeros((16,), jnp.float32)

    pltpu.sync_copy(dst_hbm.at[wid], dst_v)
    one16 = jnp.full((16,), 1.0, jnp.float32)

    @pl.loop(0, E_PER_W // 16)
    def _(k):
        idx = dst_v[pl.ds(k * 16, 16)]
        plsc.addupdate_scatter(hist, [idx], one16)

    pltpu.sync_copy(hist, out_hbm.at[wid])


def _make_sc_aggregate(d):
    @functools.partial(
        pl.kernel,
        mesh=_sc_mesh(),
        compiler_params=pltpu.CompilerParams(use_tc_tiling_on_sc=False),
        out_type=jax.ShapeDtypeStruct((NC * N, d), jnp.float32),
        scratch_types=[
            pltpu.VMEM((ROWS_PER_WORKER, CHUNK), jnp.int32),
            pltpu.VMEM((ROWS_PER_WORKER, CHUNK), jnp.int32),
            pltpu.VMEM((CHUNK, d), jnp.float32),
            pltpu.VMEM((CHUNK, d), jnp.float32),
            pltpu.VMEM_SHARED((N_PAD, d), jnp.float32),
            pltpu.SemaphoreType.DMA,
            pltpu.SemaphoreType.DMA,
        ],
    )
    def agg(y_hbm, src_hbm, dst_hbm, out_hbm, src_v, dst_v, rows_a, rows_b,
            acc_sh, sem_a, sem_b):
        cid = lax.axis_index("c")
        sid = lax.axis_index("s")
        wid = cid * NS + sid
        r0 = sid * TILE_ROWS
        pltpu.sync_copy(y_hbm.at[pl.ds(r0, TILE_ROWS)],
                        acc_sh.at[pl.ds(r0, TILE_ROWS)])

        @pl.when(sid == NS - 1)
        def _():
            pltpu.sync_copy(y_hbm.at[pl.ds(REM_R0, REM_ROWS + 8)],
                            acc_sh.at[pl.ds(REM_R0, REM_ROWS + 8)])

        pltpu.sync_copy(src_hbm.at[wid], src_v)
        pltpu.sync_copy(dst_hbm.at[wid], dst_v)
        plsc.subcore_barrier()

        bufs = (rows_a, rows_b)
        sems = (sem_a, sem_b)

        def gather(j, b):
            pltpu.async_copy(y_hbm.at[src_v.at[j]], bufs[b], sems[b])

        def gwait(b):
            pltpu.make_async_copy(y_hbm.at[src_v.at[0]], bufs[b],
                                  sems[b]).wait()

        gather(0, 0)

        @pl.loop(0, ROWS_PER_WORKER // 2)
        def _(i):
            j0 = 2 * i
            gather(j0 + 1, 1)
            gwait(0)
            pltpu.sync_copy(bufs[0], acc_sh.at[dst_v.at[j0]], add=True)
            gather(lax.rem(j0 + 2, ROWS_PER_WORKER), 0)
            gwait(1)
            pltpu.sync_copy(bufs[1], acc_sh.at[dst_v.at[j0 + 1]], add=True)

        gwait(0)

        plsc.subcore_barrier()
        pltpu.sync_copy(acc_sh.at[pl.ds(r0, TILE_ROWS)],
                        out_hbm.at[pl.ds(cid * N + r0, TILE_ROWS)])

        @pl.when(sid == NS - 1)
        def _():
            pltpu.sync_copy(acc_sh.at[pl.ds(REM_R0, REM_ROWS)],
                            out_hbm.at[pl.ds(cid * N + REM_R0, REM_ROWS)])

    return agg


_sc_agg_64 = _make_sc_aggregate(64)
_sc_agg_cls = _make_sc_aggregate(CLS)


def _row_mask(shape):
    base = pl.program_id(0) * BLK
    rows = base + lax.broadcasted_iota(jnp.int32, shape, 0)
    return rows < N


def _tc1_body(dp_ref, x_ref, w1_ref, y1a_ref, y1b_ref, dinv_ref):
    deg = jnp.sum(dp_ref[...], axis=0, keepdims=True) + 1.0
    dinv = lax.rsqrt(deg).reshape(BLK, 1)
    xw = jnp.dot(x_ref[...], w1_ref[...], preferred_element_type=jnp.float32)
    y1 = jnp.where(_row_mask((BLK, 1)), dinv * xw, 0.0)
    y1a_ref[...] = y1[:, :64]
    y1b_ref[...] = y1[:, 64:]
    dinv_ref[...] = jnp.broadcast_to(dinv, dinv_ref.shape)


_tc1 = pl.pallas_call(
    _tc1_body,
    grid=(pl.cdiv(N, BLK),),
    in_specs=[
        pl.BlockSpec((NW, BLK), lambda i: (0, i)),
        pl.BlockSpec((BLK, F_IN), lambda i: (i, 0)),
        pl.BlockSpec((F_IN, HID), lambda i: (0, 0)),
    ],
    out_specs=[
        pl.BlockSpec((BLK, 64), lambda i: (i, 0)),
        pl.BlockSpec((BLK, 64), lambda i: (i, 0)),
        pl.BlockSpec((BLK, 8), lambda i: (i, 0)),
    ],
    out_shape=[
        jax.ShapeDtypeStruct((N_PAD, 64), jnp.float32),
        jax.ShapeDtypeStruct((N_PAD, 64), jnp.float32),
        jax.ShapeDtypeStruct((N, 8), jnp.float32),
    ],
)


def _tc2_body(paa_ref, pab_ref, pba_ref, pbb_ref, ya_ref, yb_ref,
              dinv_ref, b1_ref, w2_ref, y2_ref):
    dinv = dinv_ref[:, :1]
    a1 = jnp.concatenate(
        [paa_ref[...] + pab_ref[...] - ya_ref[...],
         pba_ref[...] + pbb_ref[...] - yb_ref[...]], axis=1)
    h = jnp.maximum(dinv * a1 + b1_ref[...], 0.0)
    y2 = dinv * jnp.dot(h, w2_ref[...], preferred_element_type=jnp.float32)
    y2_ref[...] = jnp.where(_row_mask((BLK, 1)), y2, 0.0)


_tc2 = pl.pallas_call(
    _tc2_body,
    grid=(pl.cdiv(N, BLK),),
    in_specs=[
        pl.BlockSpec((BLK, 64), lambda i: (i, 0)),
        pl.BlockSpec((BLK, 64), lambda i: (i, 0)),
        pl.BlockSpec((BLK, 64), lambda i: (i, 0)),
        pl.BlockSpec((BLK, 64), lambda i: (i, 0)),
        pl.BlockSpec((BLK, 64), lambda i: (i, 0)),
        pl.BlockSpec((BLK, 64), lambda i: (i, 0)),
        pl.BlockSpec((BLK, 8), lambda i: (i, 0)),
        pl.BlockSpec((1, HID), lambda i: (0, 0)),
        pl.BlockSpec((HID, CLS), lambda i: (0, 0)),
    ],
    out_specs=pl.BlockSpec((BLK, CLS), lambda i: (i, 0)),
    out_shape=jax.ShapeDtypeStruct((N_PAD, CLS), jnp.float32),
)


def _tc3_body(pa_ref, pb_ref, y2_ref, dinv_ref, b2_ref, out_ref):
    dinv = dinv_ref[:, :1]
    z = dinv * (pa_ref[...] + pb_ref[...] - y2_ref[...]) + b2_ref[...]
    m = jnp.max(z, axis=1, keepdims=True)
    lse = m + jnp.log(jnp.sum(jnp.exp(z - m), axis=1, keepdims=True))
    out_ref[...] = z - lse


_tc3 = pl.pallas_call(
    _tc3_body,
    grid=(pl.cdiv(N, BLK),),
    in_specs=[
        pl.BlockSpec((BLK, CLS), lambda i: (i, 0)),
        pl.BlockSpec((BLK, CLS), lambda i: (i, 0)),
        pl.BlockSpec((BLK, CLS), lambda i: (i, 0)),
        pl.BlockSpec((BLK, 8), lambda i: (i, 0)),
        pl.BlockSpec((1, CLS), lambda i: (0, 0)),
    ],
    out_specs=pl.BlockSpec((BLK, CLS), lambda i: (i, 0)),
    out_shape=jax.ShapeDtypeStruct((N, CLS), jnp.float32),
)


def kernel(x, edge_index, W1, b1, W2, b2):
    ei = edge_index.astype(jnp.int32)
    pad_src = jnp.full((E_PAD - E,), N, jnp.int32)
    pad_dst = jnp.arange(E_PAD - E, dtype=jnp.int32)
    src3d = jnp.concatenate([ei[0], pad_src]).reshape(
        NW, ROWS_PER_WORKER, CHUNK)
    dst3d = jnp.concatenate([ei[1], pad_dst]).reshape(
        NW, ROWS_PER_WORKER, CHUNK)
    dst2d = jnp.concatenate([ei[1], pad_src]).reshape(NW, E_PER_W)

    degp = _sc_degree(dst2d)
    y1a, y1b, dinv8 = _tc1(degp, x, W1)
    p1a = _sc_agg_64(y1a, src3d, dst3d)
    p1b = _sc_agg_64(y1b, src3d, dst3d)
    y2 = _tc2(p1a[:N], p1a[N:], p1b[:N], p1b[N:], y1a, y1b, dinv8,
              b1.reshape(1, HID), W2)
    p2 = _sc_agg_cls(y2, src3d, dst3d)
    return _tc3(p2[:N], p2[N:], y2, dinv8, b2.reshape(1, CLS))


# speedup vs baseline: 1.8222x; 1.6470x over previous
import functools

import jax
import jax.numpy as jnp
from jax import lax
from jax.experimental import pallas as pl
from jax.experimental.pallas import tpu as pltpu
from jax.experimental.pallas import tpu_sc as plsc

N = 10000
E = 320000
F_IN = 128
HID = 128
CLS = 40

NC = 2
NS = 16
NW = NC * NS

CHUNK = 128
ROWS_PER_WORKER = 80
E_PAD = NW * ROWS_PER_WORKER * CHUNK
N_PAD = N + 8

TILE_ROWS = 624
REM_ROWS = N - TILE_ROWS * NS
REM_R0 = TILE_ROWS * NS

BLK = 1024


def _sc_mesh():
    return plsc.VectorSubcoreMesh(core_axis_name="c", subcore_axis_name="s")


E_PER_W = E_PAD // NW
HIST_PAD = 10240


@functools.partial(
    pl.kernel,
    mesh=_sc_mesh(),
    compiler_params=pltpu.CompilerParams(use_tc_tiling_on_sc=False,
                                         needs_layout_passes=False),
    out_type=jax.ShapeDtypeStruct((NW, HIST_PAD), jnp.float32),
    scratch_types=[
        pltpu.VMEM((E_PER_W,), jnp.int32),
        pltpu.VMEM((HIST_PAD,), jnp.float32),
    ],
)
def _sc_degree(dst_hbm, out_hbm, dst_v, hist):
    cid = lax.axis_index("c")
    sid = lax.axis_index("s")
    wid = cid * NS + sid

    @pl.loop(0, HIST_PAD // 16)
    def _(i):
        hist[pl.ds(i * 16, 16)] = jnp.zeros((16,), jnp.float32)

    pltpu.sync_copy(dst_hbm.at[wid], dst_v)
    one16 = jnp.full((16,), 1.0, jnp.float32)

    @pl.loop(0, E_PER_W // 16)
    def _(k):
        idx = dst_v[pl.ds(k * 16, 16)]
        plsc.addupdate_scatter(hist, [idx], one16)

    pltpu.sync_copy(hist, out_hbm.at[wid])


def _make_sc_aggregate(d):
    @functools.partial(
        pl.kernel,
        mesh=_sc_mesh(),
        compiler_params=pltpu.CompilerParams(use_tc_tiling_on_sc=False),
        out_type=jax.ShapeDtypeStruct((NC * N, d), jnp.float32),
        scratch_types=[
            pltpu.VMEM((ROWS_PER_WORKER, CHUNK), jnp.int32),
            pltpu.VMEM((ROWS_PER_WORKER, CHUNK), jnp.int32),
            pltpu.VMEM((CHUNK, d), jnp.float32),
            pltpu.VMEM((CHUNK, d), jnp.float32),
            pltpu.VMEM_SHARED((N_PAD, d), jnp.float32),
            pltpu.SemaphoreType.DMA,
            pltpu.SemaphoreType.DMA,
        ],
    )
    def agg(y_hbm, src_hbm, dst_hbm, out_hbm, src_v, dst_v, rows_a, rows_b,
            acc_sh, sem_a, sem_b):
        cid = lax.axis_index("c")
        sid = lax.axis_index("s")
        wid = cid * NS + sid
        r0 = sid * TILE_ROWS
        pltpu.sync_copy(y_hbm.at[pl.ds(r0, TILE_ROWS)],
                        acc_sh.at[pl.ds(r0, TILE_ROWS)])

        @pl.when(sid == NS - 1)
        def _():
            pltpu.sync_copy(y_hbm.at[pl.ds(REM_R0, REM_ROWS + 8)],
                            acc_sh.at[pl.ds(REM_R0, REM_ROWS + 8)])

        pltpu.sync_copy(src_hbm.at[wid], src_v)
        pltpu.sync_copy(dst_hbm.at[wid], dst_v)
        plsc.subcore_barrier()

        bufs = (rows_a, rows_b)
        sems = (sem_a, sem_b)

        def gather(j, b):
            pltpu.async_copy(y_hbm.at[src_v.at[j]], bufs[b], sems[b])

        def gwait(b):
            pltpu.make_async_copy(y_hbm.at[src_v.at[0]], bufs[b],
                                  sems[b]).wait()

        gather(0, 0)

        @pl.loop(0, ROWS_PER_WORKER // 2)
        def _(i):
            j0 = 2 * i
            gather(j0 + 1, 1)
            gwait(0)
            pltpu.sync_copy(bufs[0], acc_sh.at[dst_v.at[j0]], add=True)
            gather(lax.rem(j0 + 2, ROWS_PER_WORKER), 0)
            gwait(1)
            pltpu.sync_copy(bufs[1], acc_sh.at[dst_v.at[j0 + 1]], add=True)

        gwait(0)

        plsc.subcore_barrier()
        pltpu.sync_copy(acc_sh.at[pl.ds(r0, TILE_ROWS)],
                        out_hbm.at[pl.ds(cid * N + r0, TILE_ROWS)])

        @pl.when(sid == NS - 1)
        def _():
            pltpu.sync_copy(acc_sh.at[pl.ds(REM_R0, REM_ROWS)],
                            out_hbm.at[pl.ds(cid * N + REM_R0, REM_ROWS)])

    return agg


_sc_agg_64 = _make_sc_aggregate(64)
_sc_agg_cls = _make_sc_aggregate(CLS)


def _row_mask(shape):
    base = pl.program_id(0) * BLK
    rows = base + lax.broadcasted_iota(jnp.int32, shape, 0)
    return rows < N


def _tc1_body(dp_ref, x_ref, w1_ref, y1a_ref, y1b_ref, dinv_ref):
    deg = jnp.sum(dp_ref[...], axis=0, keepdims=True) + 1.0
    dinv = lax.rsqrt(deg).reshape(BLK, 1)
    xw = jnp.dot(x_ref[...], w1_ref[...], preferred_element_type=jnp.float32)
    y1 = jnp.where(_row_mask((BLK, 1)), dinv * xw, 0.0)
    y1a_ref[...] = y1[:, :64]
    y1b_ref[...] = y1[:, 64:]
    dinv_ref[...] = jnp.broadcast_to(dinv, dinv_ref.shape)


_tc1 = pl.pallas_call(
    _tc1_body,
    grid=(pl.cdiv(N, BLK),),
    in_specs=[
        pl.BlockSpec((NW, BLK), lambda i: (0, i)),
        pl.BlockSpec((BLK, F_IN), lambda i: (i, 0)),
        pl.BlockSpec((F_IN, HID), lambda i: (0, 0)),
    ],
    out_specs=[
        pl.BlockSpec((BLK, 64), lambda i: (i, 0)),
        pl.BlockSpec((BLK, 64), lambda i: (i, 0)),
        pl.BlockSpec((BLK, 8), lambda i: (i, 0)),
    ],
    out_shape=[
        jax.ShapeDtypeStruct((N_PAD, 64), jnp.float32),
        jax.ShapeDtypeStruct((N_PAD, 64), jnp.float32),
        jax.ShapeDtypeStruct((N, 8), jnp.float32),
    ],
)


def _tc2_body(paa_ref, pab_ref, pba_ref, pbb_ref, ya_ref, yb_ref,
              dinv_ref, b1_ref, w2_ref, y2_ref):
    dinv = dinv_ref[:, :1]
    a1 = jnp.concatenate(
        [paa_ref[...] + pab_ref[...] - ya_ref[...],
         pba_ref[...] + pbb_ref[...] - yb_ref[...]], axis=1)
    h = jnp.maximum(dinv * a1 + b1_ref[...], 0.0)
    y2 = dinv * jnp.dot(h, w2_ref[...], preferred_element_type=jnp.float32)
    y2_ref[...] = jnp.where(_row_mask((BLK, 1)), y2, 0.0)


_tc2 = pl.pallas_call(
    _tc2_body,
    grid=(pl.cdiv(N, BLK),),
    in_specs=[
        pl.BlockSpec((BLK, 64), lambda i: (i, 0)),
        pl.BlockSpec((BLK, 64), lambda i: (i, 0)),
        pl.BlockSpec((BLK, 64), lambda i: (i, 0)),
        pl.BlockSpec((BLK, 64), lambda i: (i, 0)),
        pl.BlockSpec((BLK, 64), lambda i: (i, 0)),
        pl.BlockSpec((BLK, 64), lambda i: (i, 0)),
        pl.BlockSpec((BLK, 8), lambda i: (i, 0)),
        pl.BlockSpec((1, HID), lambda i: (0, 0)),
        pl.BlockSpec((HID, CLS), lambda i: (0, 0)),
    ],
    out_specs=pl.BlockSpec((BLK, CLS), lambda i: (i, 0)),
    out_shape=jax.ShapeDtypeStruct((N_PAD, CLS), jnp.float32),
)


def _tc3_body(pa_ref, pb_ref, y2_ref, dinv_ref, b2_ref, out_ref):
    dinv = dinv_ref[:, :1]
    z = dinv * (pa_ref[...] + pb_ref[...] - y2_ref[...]) + b2_ref[...]
    m = jnp.max(z, axis=1, keepdims=True)
    lse = m + jnp.log(jnp.sum(jnp.exp(z - m), axis=1, keepdims=True))
    out_ref[...] = z - lse


_tc3 = pl.pallas_call(
    _tc3_body,
    grid=(pl.cdiv(N, BLK),),
    in_specs=[
        pl.BlockSpec((BLK, CLS), lambda i: (i, 0)),
        pl.BlockSpec((BLK, CLS), lambda i: (i, 0)),
        pl.BlockSpec((BLK, CLS), lambda i: (i, 0)),
        pl.BlockSpec((BLK, 8), lambda i: (i, 0)),
        pl.BlockSpec((1, CLS), lambda i: (0, 0)),
    ],
    out_specs=pl.BlockSpec((BLK, CLS), lambda i: (i, 0)),
    out_shape=jax.ShapeDtypeStruct((N, CLS), jnp.float32),
)


def kernel(x, edge_index, W1, b1, W2, b2):
    ei = edge_index.astype(jnp.int32)
    ppw = E_PER_W - E // NW
    srcw = ei[0].reshape(NW, E // NW)
    dstw = ei[1].reshape(NW, E // NW)
    pad_src = N + jnp.tile(jnp.arange(8, dtype=jnp.int32), (NW, ppw // 8))
    pad_dst = jnp.arange(NW * ppw, dtype=jnp.int32).reshape(NW, ppw)
    src3d = jnp.concatenate([srcw, pad_src], axis=1).reshape(
        NW, ROWS_PER_WORKER, CHUNK)
    dst3d = jnp.concatenate([dstw, pad_dst], axis=1).reshape(
        NW, ROWS_PER_WORKER, CHUNK)
    dst2d = jnp.concatenate(
        [dstw, jnp.full((NW, ppw), N, jnp.int32)], axis=1)

    degp = _sc_degree(dst2d)
    y1a, y1b, dinv8 = _tc1(degp, x, W1)
    p1a = _sc_agg_64(y1a, src3d, dst3d)
    p1b = _sc_agg_64(y1b, src3d, dst3d)
    y2 = _tc2(p1a[:N], p1a[N:], p1b[:N], p1b[N:], y1a, y1b, dinv8,
              b1.reshape(1, HID), W2)
    p2 = _sc_agg_cls(y2, src3d, dst3d)
    return _tc3(p2[:N], p2[N:], y2, dinv8, b2.reshape(1, CLS))


# speedup vs baseline: 1.9484x; 1.0692x over previous
import functools

import jax
import jax.numpy as jnp
from jax import lax
from jax.experimental import pallas as pl
from jax.experimental.pallas import tpu as pltpu
from jax.experimental.pallas import tpu_sc as plsc

N = 10000
E = 320000
F_IN = 128
HID = 128
CLS = 40

NC = 2
NS = 16
NW = NC * NS

CHUNK = 128
ROWS_PER_WORKER = 80
E_PAD = NW * ROWS_PER_WORKER * CHUNK
N_PAD = N + 8

TILE_ROWS = 624
REM_ROWS = N - TILE_ROWS * NS
REM_R0 = TILE_ROWS * NS

BLK = 1024


def _sc_mesh():
    return plsc.VectorSubcoreMesh(core_axis_name="c", subcore_axis_name="s")


E_PER_W = E_PAD // NW
HIST_PAD = 10240


@functools.partial(
    pl.kernel,
    mesh=_sc_mesh(),
    compiler_params=pltpu.CompilerParams(use_tc_tiling_on_sc=False,
                                         needs_layout_passes=False),
    out_type=jax.ShapeDtypeStruct((NW, HIST_PAD), jnp.float32),
    scratch_types=[
        pltpu.VMEM((E_PER_W,), jnp.int32),
        pltpu.VMEM((HIST_PAD,), jnp.float32),
    ],
)
def _sc_degree(dst_hbm, out_hbm, dst_v, hist):
    cid = lax.axis_index("c")
    sid = lax.axis_index("s")
    wid = cid * NS + sid

    @pl.loop(0, HIST_PAD // 16)
    def _(i):
        hist[pl.ds(i * 16, 16)] = jnp.zeros((16,), jnp.float32)

    pltpu.sync_copy(dst_hbm.at[wid], dst_v)
    one16 = jnp.full((16,), 1.0, jnp.float32)

    @pl.loop(0, E_PER_W // 16)
    def _(k):
        idx = dst_v[pl.ds(k * 16, 16)]
        plsc.addupdate_scatter(hist, [idx], one16)

    pltpu.sync_copy(hist, out_hbm.at[wid])


def _make_sc_aggregate(d, nbuf):
    row_scratch = [pltpu.VMEM((CHUNK, d), jnp.float32)] * nbuf
    sem_scratch = [pltpu.SemaphoreType.DMA] * nbuf

    @functools.partial(
        pl.kernel,
        mesh=_sc_mesh(),
        compiler_params=pltpu.CompilerParams(use_tc_tiling_on_sc=False),
        out_type=jax.ShapeDtypeStruct((NC * N, d), jnp.float32),
        scratch_types=[
            pltpu.VMEM((ROWS_PER_WORKER, CHUNK), jnp.int32),
            pltpu.VMEM((ROWS_PER_WORKER, CHUNK), jnp.int32),
        ] + row_scratch + [
            pltpu.VMEM_SHARED((N, d), jnp.float32),
        ] + sem_scratch,
    )
    def agg(y_hbm, src_hbm, dst_hbm, out_hbm, src_v, dst_v, *rest):
        bufs = rest[:nbuf]
        acc_sh = rest[nbuf]
        sems = rest[nbuf + 1:]
        cid = lax.axis_index("c")
        sid = lax.axis_index("s")
        wid = cid * NS + sid
        r0 = sid * TILE_ROWS
        pltpu.sync_copy(y_hbm.at[pl.ds(r0, TILE_ROWS)],
                        acc_sh.at[pl.ds(r0, TILE_ROWS)])

        @pl.when(sid == NS - 1)
        def _():
            pltpu.sync_copy(y_hbm.at[pl.ds(REM_R0, REM_ROWS)],
                            acc_sh.at[pl.ds(REM_R0, REM_ROWS)])

        pltpu.sync_copy(src_hbm.at[wid], src_v)
        pltpu.sync_copy(dst_hbm.at[wid], dst_v)
        plsc.subcore_barrier()

        def gather(j, b):
            pltpu.async_copy(y_hbm.at[src_v.at[j]], bufs[b], sems[b])

        def gwait(b):
            pltpu.make_async_copy(y_hbm.at[src_v.at[0]], bufs[b],
                                  sems[b]).wait()

        if nbuf == 2:
            gather(0, 0)

            @pl.loop(0, ROWS_PER_WORKER // 2)
            def _(i):
                j0 = 2 * i
                gather(j0 + 1, 1)
                gwait(0)
                pltpu.sync_copy(bufs[0], acc_sh.at[dst_v.at[j0]], add=True)
                gather(lax.rem(j0 + 2, ROWS_PER_WORKER), 0)
                gwait(1)
                pltpu.sync_copy(bufs[1], acc_sh.at[dst_v.at[j0 + 1]], add=True)

            gwait(0)
        else:
            @pl.loop(0, ROWS_PER_WORKER)
            def _(j):
                gather(j, 0)
                gwait(0)
                pltpu.sync_copy(bufs[0], acc_sh.at[dst_v.at[j]], add=True)

        plsc.subcore_barrier()
        pltpu.sync_copy(acc_sh.at[pl.ds(r0, TILE_ROWS)],
                        out_hbm.at[pl.ds(cid * N + r0, TILE_ROWS)])

        @pl.when(sid == NS - 1)
        def _():
            pltpu.sync_copy(acc_sh.at[pl.ds(REM_R0, REM_ROWS)],
                            out_hbm.at[pl.ds(cid * N + REM_R0, REM_ROWS)])

    return agg


_sc_agg1 = _make_sc_aggregate(HID, nbuf=1)
_sc_agg2 = _make_sc_aggregate(CLS, nbuf=2)


def _row_mask(shape):
    base = pl.program_id(0) * BLK
    rows = base + lax.broadcasted_iota(jnp.int32, shape, 0)
    return rows < N


def _tc1_body(dp_ref, x_ref, w1_ref, y1_ref, dinv_ref):
    deg = jnp.sum(dp_ref[...], axis=0, keepdims=True) + 1.0
    dinv = lax.rsqrt(deg).reshape(BLK, 1)
    xw = jnp.dot(x_ref[...], w1_ref[...], preferred_element_type=jnp.float32)
    y1_ref[...] = jnp.where(_row_mask((BLK, 1)), dinv * xw, 0.0)
    dinv_ref[...] = jnp.broadcast_to(dinv, dinv_ref.shape)


_tc1 = pl.pallas_call(
    _tc1_body,
    grid=(pl.cdiv(N, BLK),),
    in_specs=[
        pl.BlockSpec((NW, BLK), lambda i: (0, i)),
        pl.BlockSpec((BLK, F_IN), lambda i: (i, 0)),
        pl.BlockSpec((F_IN, HID), lambda i: (0, 0)),
    ],
    out_specs=[
        pl.BlockSpec((BLK, HID), lambda i: (i, 0)),
        pl.BlockSpec((BLK, 8), lambda i: (i, 0)),
    ],
    out_shape=[
        jax.ShapeDtypeStruct((N_PAD, HID), jnp.float32),
        jax.ShapeDtypeStruct((N, 8), jnp.float32),
    ],
)


def _tc2_body(pa_ref, pb_ref, y1_ref, dinv_ref, b1_ref, w2_ref, y2_ref):
    dinv = dinv_ref[:, :1]
    a1 = pa_ref[...] + pb_ref[...] - y1_ref[...]
    h = jnp.maximum(dinv * a1 + b1_ref[...], 0.0)
    y2 = dinv * jnp.dot(h, w2_ref[...], preferred_element_type=jnp.float32)
    y2_ref[...] = jnp.where(_row_mask((BLK, 1)), y2, 0.0)


_tc2 = pl.pallas_call(
    _tc2_body,
    grid=(pl.cdiv(N, BLK),),
    in_specs=[
        pl.BlockSpec((BLK, HID), lambda i: (i, 0)),
        pl.BlockSpec((BLK, HID), lambda i: (i, 0)),
        pl.BlockSpec((BLK, HID), lambda i: (i, 0)),
        pl.BlockSpec((BLK, 8), lambda i: (i, 0)),
        pl.BlockSpec((1, HID), lambda i: (0, 0)),
        pl.BlockSpec((HID, CLS), lambda i: (0, 0)),
    ],
    out_specs=pl.BlockSpec((BLK, CLS), lambda i: (i, 0)),
    out_shape=jax.ShapeDtypeStruct((N_PAD, CLS), jnp.float32),
)


def _tc3_body(pa_ref, pb_ref, y2_ref, dinv_ref, b2_ref, out_ref):
    dinv = dinv_ref[:, :1]
    z = dinv * (pa_ref[...] + pb_ref[...] - y2_ref[...]) + b2_ref[...]
    m = jnp.max(z, axis=1, keepdims=True)
    lse = m + jnp.log(jnp.sum(jnp.exp(z - m), axis=1, keepdims=True))
    out_ref[...] = z - lse


_tc3 = pl.pallas_call(
    _tc3_body,
    grid=(pl.cdiv(N, BLK),),
    in_specs=[
        pl.BlockSpec((BLK, CLS), lambda i: (i, 0)),
        pl.BlockSpec((BLK, CLS), lambda i: (i, 0)),
        pl.BlockSpec((BLK, CLS), lambda i: (i, 0)),
        pl.BlockSpec((BLK, 8), lambda i: (i, 0)),
        pl.BlockSpec((1, CLS), lambda i: (0, 0)),
    ],
    out_specs=pl.BlockSpec((BLK, CLS), lambda i: (i, 0)),
    out_shape=jax.ShapeDtypeStruct((N, CLS), jnp.float32),
)


def kernel(x, edge_index, W1, b1, W2, b2):
    ei = edge_index.astype(jnp.int32)
    ppw = E_PER_W - E // NW
    srcw = ei[0].reshape(NW, E // NW)
    dstw = ei[1].reshape(NW, E // NW)
    pad_src = N + jnp.tile(jnp.arange(8, dtype=jnp.int32), (NW, ppw // 8))
    pad_dst = jnp.arange(NW * ppw, dtype=jnp.int32).reshape(NW, ppw)
    src3d = jnp.concatenate([srcw, pad_src], axis=1).reshape(
        NW, ROWS_PER_WORKER, CHUNK)
    dst3d = jnp.concatenate([dstw, pad_dst], axis=1).reshape(
        NW, ROWS_PER_WORKER, CHUNK)
    dst2d = jnp.concatenate(
        [dstw, jnp.full((NW, ppw), N, jnp.int32)], axis=1)

    degp = _sc_degree(dst2d)
    y1, dinv8 = _tc1(degp, x, W1)
    p1 = _sc_agg1(y1, src3d, dst3d)
    y2 = _tc2(p1[:N], p1[N:], y1, dinv8,
              b1.reshape(1, HID), W2)
    p2 = _sc_agg2(y2, src3d, dst3d)
    return _tc3(p2[:N], p2[N:], y2, dinv8, b2.reshape(1, CLS))


# speedup vs baseline: 2.1695x; 1.1135x over previous
import functools

import jax
import jax.numpy as jnp
from jax import lax
from jax.experimental import pallas as pl
from jax.experimental.pallas import tpu as pltpu
from jax.experimental.pallas import tpu_sc as plsc

N = 10000
E = 320000
F_IN = 128
HID = 128
CLS = 40

NC = 2
NS = 16
NW = NC * NS

CHUNK = 128
ROWS_PER_WORKER = 80
E_PAD = NW * ROWS_PER_WORKER * CHUNK
N_PAD = N + 8

TILE_ROWS = 624
REM_ROWS = N - TILE_ROWS * NS
REM_R0 = TILE_ROWS * NS

BLK = 1024


def _sc_mesh():
    return plsc.VectorSubcoreMesh(core_axis_name="c", subcore_axis_name="s")


E_PER_W = E_PAD // NW
HIST_PAD = 10240


@functools.partial(
    pl.kernel,
    mesh=_sc_mesh(),
    compiler_params=pltpu.CompilerParams(use_tc_tiling_on_sc=False,
                                         needs_layout_passes=False),
    out_type=jax.ShapeDtypeStruct((NW, HIST_PAD), jnp.float32),
    scratch_types=[
        pltpu.VMEM((E_PER_W,), jnp.int32),
        pltpu.VMEM((HIST_PAD,), jnp.float32),
    ],
)
def _sc_degree(dst_hbm, out_hbm, dst_v, hist):
    cid = lax.axis_index("c")
    sid = lax.axis_index("s")
    wid = cid * NS + sid

    @pl.loop(0, HIST_PAD // 16)
    def _(i):
        hist[pl.ds(i * 16, 16)] = jnp.zeros((16,), jnp.float32)

    pltpu.sync_copy(dst_hbm.at[wid], dst_v)
    one16 = jnp.full((16,), 1.0, jnp.float32)

    @pl.loop(0, E_PER_W // 16)
    def _(k):
        idx = dst_v[pl.ds(k * 16, 16)]
        plsc.addupdate_scatter(hist, [idx], one16)

    pltpu.sync_copy(hist, out_hbm.at[wid])


def _make_sc_aggregate(d, ch):
    rpw = E_PER_W // ch

    @functools.partial(
        pl.kernel,
        mesh=_sc_mesh(),
        compiler_params=pltpu.CompilerParams(use_tc_tiling_on_sc=False),
        out_type=jax.ShapeDtypeStruct((NC * N, d), jnp.float32),
        scratch_types=[
            pltpu.VMEM((rpw, ch), jnp.int32),
            pltpu.VMEM((rpw, ch), jnp.int32),
            pltpu.VMEM((2 * ch, d), jnp.float32),
            pltpu.VMEM_SHARED((N, d), jnp.float32),
            pltpu.SemaphoreType.DMA,
            pltpu.SemaphoreType.DMA,
        ],
    )
    def agg(y_hbm, src_hbm, dst_hbm, out_hbm, src_v, dst_v, rows_v, acc_sh,
            sem_a, sem_b):
        bufs = (rows_v.at[pl.ds(0, ch)], rows_v.at[pl.ds(ch, ch)])
        sems = (sem_a, sem_b)
        cid = lax.axis_index("c")
        sid = lax.axis_index("s")
        wid = cid * NS + sid
        r0 = sid * TILE_ROWS
        pltpu.sync_copy(y_hbm.at[pl.ds(r0, TILE_ROWS)],
                        acc_sh.at[pl.ds(r0, TILE_ROWS)])

        @pl.when(sid == NS - 1)
        def _():
            pltpu.sync_copy(y_hbm.at[pl.ds(REM_R0, REM_ROWS)],
                            acc_sh.at[pl.ds(REM_R0, REM_ROWS)])

        pltpu.sync_copy(src_hbm.at[wid], src_v)
        pltpu.sync_copy(dst_hbm.at[wid], dst_v)
        plsc.subcore_barrier()

        def gather(j, b):
            pltpu.async_copy(y_hbm.at[src_v.at[j]], bufs[b], sems[b])

        def gwait(b):
            pltpu.make_async_copy(y_hbm.at[src_v.at[0]], bufs[b],
                                  sems[b]).wait()

        gather(0, 0)

        @pl.loop(0, rpw // 2)
        def _(i):
            j0 = 2 * i
            gather(j0 + 1, 1)
            gwait(0)
            pltpu.sync_copy(bufs[0], acc_sh.at[dst_v.at[j0]], add=True)
            gather(lax.rem(j0 + 2, rpw), 0)
            gwait(1)
            pltpu.sync_copy(bufs[1], acc_sh.at[dst_v.at[j0 + 1]], add=True)

        gwait(0)

        plsc.subcore_barrier()
        pltpu.sync_copy(acc_sh.at[pl.ds(r0, TILE_ROWS)],
                        out_hbm.at[pl.ds(cid * N + r0, TILE_ROWS)])

        @pl.when(sid == NS - 1)
        def _():
            pltpu.sync_copy(acc_sh.at[pl.ds(REM_R0, REM_ROWS)],
                            out_hbm.at[pl.ds(cid * N + REM_R0, REM_ROWS)])

    return agg


_sc_agg1 = _make_sc_aggregate(HID, ch=64)
_sc_agg2 = _make_sc_aggregate(CLS, ch=128)


def _row_mask(shape):
    base = pl.program_id(0) * BLK
    rows = base + lax.broadcasted_iota(jnp.int32, shape, 0)
    return rows < N


def _tc1_body(dp_ref, x_ref, w1_ref, y1_ref, dinv_ref):
    deg = jnp.sum(dp_ref[...], axis=0, keepdims=True) + 1.0
    dinv = lax.rsqrt(deg).reshape(BLK, 1)
    xw = jnp.dot(x_ref[...], w1_ref[...], preferred_element_type=jnp.float32)
    y1_ref[...] = jnp.where(_row_mask((BLK, 1)), dinv * xw, 0.0)
    dinv_ref[...] = jnp.broadcast_to(dinv, dinv_ref.shape)


_tc1 = pl.pallas_call(
    _tc1_body,
    grid=(pl.cdiv(N, BLK),),
    in_specs=[
        pl.BlockSpec((NW, BLK), lambda i: (0, i)),
        pl.BlockSpec((BLK, F_IN), lambda i: (i, 0)),
        pl.BlockSpec((F_IN, HID), lambda i: (0, 0)),
    ],
    out_specs=[
        pl.BlockSpec((BLK, HID), lambda i: (i, 0)),
        pl.BlockSpec((BLK, 8), lambda i: (i, 0)),
    ],
    out_shape=[
        jax.ShapeDtypeStruct((N_PAD, HID), jnp.float32),
        jax.ShapeDtypeStruct((N, 8), jnp.float32),
    ],
)


def _tc2_body(pa_ref, pb_ref, y1_ref, dinv_ref, b1_ref, w2_ref, y2_ref):
    dinv = dinv_ref[:, :1]
    a1 = pa_ref[...] + pb_ref[...] - y1_ref[...]
    h = jnp.maximum(dinv * a1 + b1_ref[...], 0.0)
    y2 = dinv * jnp.dot(h, w2_ref[...], preferred_element_type=jnp.float32)
    y2_ref[...] = jnp.where(_row_mask((BLK, 1)), y2, 0.0)


_tc2 = pl.pallas_call(
    _tc2_body,
    grid=(pl.cdiv(N, BLK),),
    in_specs=[
        pl.BlockSpec((BLK, HID), lambda i: (i, 0)),
        pl.BlockSpec((BLK, HID), lambda i: (i, 0)),
        pl.BlockSpec((BLK, HID), lambda i: (i, 0)),
        pl.BlockSpec((BLK, 8), lambda i: (i, 0)),
        pl.BlockSpec((1, HID), lambda i: (0, 0)),
        pl.BlockSpec((HID, CLS), lambda i: (0, 0)),
    ],
    out_specs=pl.BlockSpec((BLK, CLS), lambda i: (i, 0)),
    out_shape=jax.ShapeDtypeStruct((N_PAD, CLS), jnp.float32),
)


def _tc3_body(pa_ref, pb_ref, y2_ref, dinv_ref, b2_ref, out_ref):
    dinv = dinv_ref[:, :1]
    z = dinv * (pa_ref[...] + pb_ref[...] - y2_ref[...]) + b2_ref[...]
    m = jnp.max(z, axis=1, keepdims=True)
    lse = m + jnp.log(jnp.sum(jnp.exp(z - m), axis=1, keepdims=True))
    out_ref[...] = z - lse


_tc3 = pl.pallas_call(
    _tc3_body,
    grid=(pl.cdiv(N, BLK),),
    in_specs=[
        pl.BlockSpec((BLK, CLS), lambda i: (i, 0)),
        pl.BlockSpec((BLK, CLS), lambda i: (i, 0)),
        pl.BlockSpec((BLK, CLS), lambda i: (i, 0)),
        pl.BlockSpec((BLK, 8), lambda i: (i, 0)),
        pl.BlockSpec((1, CLS), lambda i: (0, 0)),
    ],
    out_specs=pl.BlockSpec((BLK, CLS), lambda i: (i, 0)),
    out_shape=jax.ShapeDtypeStruct((N, CLS), jnp.float32),
)


def kernel(x, edge_index, W1, b1, W2, b2):
    ei = edge_index.astype(jnp.int32)
    ppw = E_PER_W - E // NW
    srcw = ei[0].reshape(NW, E // NW)
    dstw = ei[1].reshape(NW, E // NW)
    pad_src = N + jnp.tile(jnp.arange(8, dtype=jnp.int32), (NW, ppw // 8))
    pad_dst = jnp.arange(NW * ppw, dtype=jnp.int32).reshape(NW, ppw)
    src_flat = jnp.concatenate([srcw, pad_src], axis=1)
    dst_flat = jnp.concatenate([dstw, pad_dst], axis=1)
    src3d_a = src_flat.reshape(NW, E_PER_W // 64, 64)
    dst3d_a = dst_flat.reshape(NW, E_PER_W // 64, 64)
    src3d = src_flat.reshape(NW, ROWS_PER_WORKER, CHUNK)
    dst3d = dst_flat.reshape(NW, ROWS_PER_WORKER, CHUNK)
    dst2d = jnp.concatenate(
        [dstw, jnp.full((NW, ppw), N, jnp.int32)], axis=1)

    degp = _sc_degree(dst2d)
    y1, dinv8 = _tc1(degp, x, W1)
    p1 = _sc_agg1(y1, src3d_a, dst3d_a)
    y2 = _tc2(p1[:N], p1[N:], y1, dinv8,
              b1.reshape(1, HID), W2)
    p2 = _sc_agg2(y2, src3d, dst3d)
    return _tc3(p2[:N], p2[N:], y2, dinv8, b2.reshape(1, CLS))


# speedup vs baseline: 2.2223x; 1.0243x over previous
import functools

import jax
import jax.numpy as jnp
from jax import lax
from jax.experimental import pallas as pl
from jax.experimental.pallas import tpu as pltpu
from jax.experimental.pallas import tpu_sc as plsc

N = 10000
E = 320000
F_IN = 128
HID = 128
CLS = 40

NC = 2
NS = 16
NW = NC * NS

CHUNK = 128
ROWS_PER_WORKER = 80
E_PAD = NW * ROWS_PER_WORKER * CHUNK
N_PAD = N + 8

TILE_ROWS = 624
REM_ROWS = N - TILE_ROWS * NS
REM_R0 = TILE_ROWS * NS

BLK = 1024


def _sc_mesh():
    return plsc.VectorSubcoreMesh(core_axis_name="c", subcore_axis_name="s")


E_PER_W = E_PAD // NW
HIST_PAD = 10240


@functools.partial(
    pl.kernel,
    mesh=_sc_mesh(),
    compiler_params=pltpu.CompilerParams(use_tc_tiling_on_sc=False,
                                         needs_layout_passes=False),
    out_type=jax.ShapeDtypeStruct((NW, HIST_PAD), jnp.float32),
    scratch_types=[
        pltpu.VMEM((E_PER_W,), jnp.int32),
        pltpu.VMEM((HIST_PAD,), jnp.float32),
    ],
)
def _sc_degree(dst_hbm, out_hbm, dst_v, hist):
    cid = lax.axis_index("c")
    sid = lax.axis_index("s")
    wid = cid * NS + sid

    @pl.loop(0, HIST_PAD // 16)
    def _(i):
        hist[pl.ds(i * 16, 16)] = jnp.zeros((16,), jnp.float32)

    pltpu.sync_copy(dst_hbm.at[wid], dst_v)
    one16 = jnp.full((16,), 1.0, jnp.float32)

    @pl.loop(0, E_PER_W // 16)
    def _(k):
        idx = dst_v[pl.ds(k * 16, 16)]
        plsc.addupdate_scatter(hist, [idx], one16)

    pltpu.sync_copy(hist, out_hbm.at[wid])


def _make_sc_aggregate(d, ch):
    rpw = E_PER_W // ch

    @functools.partial(
        pl.kernel,
        mesh=_sc_mesh(),
        compiler_params=pltpu.CompilerParams(use_tc_tiling_on_sc=False),
        out_type=jax.ShapeDtypeStruct((NC * N, d), jnp.float32),
        scratch_types=[
            pltpu.VMEM((rpw, ch), jnp.int32),
            pltpu.VMEM((rpw, ch), jnp.int32),
            pltpu.VMEM((2 * ch, d), jnp.float32),
            pltpu.VMEM_SHARED((N, d), jnp.float32),
            pltpu.SemaphoreType.DMA,
            pltpu.SemaphoreType.DMA,
        ],
    )
    def agg(y_hbm, src_hbm, dst_hbm, out_hbm, src_v, dst_v, rows_v, acc_sh,
            sem_a, sem_b):
        bufs = (rows_v.at[pl.ds(0, ch)], rows_v.at[pl.ds(ch, ch)])
        sems = (sem_a, sem_b)
        cid = lax.axis_index("c")
        sid = lax.axis_index("s")
        wid = cid * NS + sid
        r0 = sid * TILE_ROWS
        pltpu.sync_copy(y_hbm.at[pl.ds(r0, TILE_ROWS)],
                        acc_sh.at[pl.ds(r0, TILE_ROWS)])

        @pl.when(sid == NS - 1)
        def _():
            pltpu.sync_copy(y_hbm.at[pl.ds(REM_R0, REM_ROWS)],
                            acc_sh.at[pl.ds(REM_R0, REM_ROWS)])

        pltpu.sync_copy(src_hbm.at[wid], src_v)
        pltpu.sync_copy(dst_hbm.at[wid], dst_v)
        plsc.subcore_barrier()

        def gather(j, b):
            pltpu.async_copy(y_hbm.at[src_v.at[j]], bufs[b], sems[b])

        def gwait(b):
            pltpu.make_async_copy(y_hbm.at[src_v.at[0]], bufs[b],
                                  sems[b]).wait()

        gather(0, 0)

        @pl.loop(0, rpw // 2)
        def _(i):
            j0 = 2 * i
            gather(j0 + 1, 1)
            gwait(0)
            pltpu.sync_copy(bufs[0], acc_sh.at[dst_v.at[j0]], add=True)
            gather(lax.rem(j0 + 2, rpw), 0)
            gwait(1)
            pltpu.sync_copy(bufs[1], acc_sh.at[dst_v.at[j0 + 1]], add=True)

        gwait(0)

        plsc.subcore_barrier()
        pltpu.sync_copy(acc_sh.at[pl.ds(r0, TILE_ROWS)],
                        out_hbm.at[pl.ds(cid * N + r0, TILE_ROWS)])

        @pl.when(sid == NS - 1)
        def _():
            pltpu.sync_copy(acc_sh.at[pl.ds(REM_R0, REM_ROWS)],
                            out_hbm.at[pl.ds(cid * N + REM_R0, REM_ROWS)])

    return agg


_sc_agg1 = _make_sc_aggregate(HID, ch=80)
_sc_agg2 = _make_sc_aggregate(CLS, ch=128)


def _row_mask(shape):
    base = pl.program_id(0) * BLK
    rows = base + lax.broadcasted_iota(jnp.int32, shape, 0)
    return rows < N


def _tc1_body(dp_ref, x_ref, w1_ref, y1_ref, dinv_ref):
    deg = jnp.sum(dp_ref[...], axis=0, keepdims=True) + 1.0
    dinv = lax.rsqrt(deg).reshape(BLK, 1)
    xw = jnp.dot(x_ref[...], w1_ref[...], preferred_element_type=jnp.float32)
    y1_ref[...] = jnp.where(_row_mask((BLK, 1)), dinv * xw, 0.0)
    dinv_ref[...] = jnp.broadcast_to(dinv, dinv_ref.shape)


_tc1 = pl.pallas_call(
    _tc1_body,
    grid=(pl.cdiv(N, BLK),),
    in_specs=[
        pl.BlockSpec((NW, BLK), lambda i: (0, i)),
        pl.BlockSpec((BLK, F_IN), lambda i: (i, 0)),
        pl.BlockSpec((F_IN, HID), lambda i: (0, 0)),
    ],
    out_specs=[
        pl.BlockSpec((BLK, HID), lambda i: (i, 0)),
        pl.BlockSpec((BLK, 8), lambda i: (i, 0)),
    ],
    out_shape=[
        jax.ShapeDtypeStruct((N_PAD, HID), jnp.float32),
        jax.ShapeDtypeStruct((N, 8), jnp.float32),
    ],
)


def _tc2_body(pa_ref, pb_ref, y1_ref, dinv_ref, b1_ref, w2_ref, y2_ref):
    dinv = dinv_ref[:, :1]
    a1 = pa_ref[...] + pb_ref[...] - y1_ref[...]
    h = jnp.maximum(dinv * a1 + b1_ref[...], 0.0)
    y2 = dinv * jnp.dot(h, w2_ref[...], preferred_element_type=jnp.float32)
    y2_ref[...] = jnp.where(_row_mask((BLK, 1)), y2, 0.0)


_tc2 = pl.pallas_call(
    _tc2_body,
    grid=(pl.cdiv(N, BLK),),
    in_specs=[
        pl.BlockSpec((BLK, HID), lambda i: (i, 0)),
        pl.BlockSpec((BLK, HID), lambda i: (i, 0)),
        pl.BlockSpec((BLK, HID), lambda i: (i, 0)),
        pl.BlockSpec((BLK, 8), lambda i: (i, 0)),
        pl.BlockSpec((1, HID), lambda i: (0, 0)),
        pl.BlockSpec((HID, CLS), lambda i: (0, 0)),
    ],
    out_specs=pl.BlockSpec((BLK, CLS), lambda i: (i, 0)),
    out_shape=jax.ShapeDtypeStruct((N_PAD, CLS), jnp.float32),
)


def _tc3_body(pa_ref, pb_ref, y2_ref, dinv_ref, b2_ref, out_ref):
    dinv = dinv_ref[:, :1]
    z = dinv * (pa_ref[...] + pb_ref[...] - y2_ref[...]) + b2_ref[...]
    m = jnp.max(z, axis=1, keepdims=True)
    lse = m + jnp.log(jnp.sum(jnp.exp(z - m), axis=1, keepdims=True))
    out_ref[...] = z - lse


_tc3 = pl.pallas_call(
    _tc3_body,
    grid=(pl.cdiv(N, BLK),),
    in_specs=[
        pl.BlockSpec((BLK, CLS), lambda i: (i, 0)),
        pl.BlockSpec((BLK, CLS), lambda i: (i, 0)),
        pl.BlockSpec((BLK, CLS), lambda i: (i, 0)),
        pl.BlockSpec((BLK, 8), lambda i: (i, 0)),
        pl.BlockSpec((1, CLS), lambda i: (0, 0)),
    ],
    out_specs=pl.BlockSpec((BLK, CLS), lambda i: (i, 0)),
    out_shape=jax.ShapeDtypeStruct((N, CLS), jnp.float32),
)


def kernel(x, edge_index, W1, b1, W2, b2):
    ei = edge_index.astype(jnp.int32)
    ppw = E_PER_W - E // NW
    srcw = ei[0].reshape(NW, E // NW)
    dstw = ei[1].reshape(NW, E // NW)
    pad_src = N + jnp.tile(jnp.arange(8, dtype=jnp.int32), (NW, ppw // 8))
    pad_dst = jnp.arange(NW * ppw, dtype=jnp.int32).reshape(NW, ppw)
    src_flat = jnp.concatenate([srcw, pad_src], axis=1)
    dst_flat = jnp.concatenate([dstw, pad_dst], axis=1)
    src3d_a = src_flat.reshape(NW, E_PER_W // 80, 80)
    dst3d_a = dst_flat.reshape(NW, E_PER_W // 80, 80)
    src3d = src_flat.reshape(NW, ROWS_PER_WORKER, CHUNK)
    dst3d = dst_flat.reshape(NW, ROWS_PER_WORKER, CHUNK)
    dst2d = jnp.concatenate(
        [dstw, jnp.full((NW, ppw), N, jnp.int32)], axis=1)

    degp = _sc_degree(dst2d)
    y1, dinv8 = _tc1(degp, x, W1)
    p1 = _sc_agg1(y1, src3d_a, dst3d_a)
    y2 = _tc2(p1[:N], p1[N:], y1, dinv8,
              b1.reshape(1, HID), W2)
    p2 = _sc_agg2(y2, src3d, dst3d)
    return _tc3(p2[:N], p2[N:], y2, dinv8, b2.reshape(1, CLS))
